# baseline (speedup 1.0000x reference)
import functools

import jax
import jax.numpy as jnp
from jax import lax
from jax.experimental import pallas as pl
from jax.experimental.pallas import tpu as pltpu
from jax.experimental.pallas import tpu_sc as plsc

B = 16384
F = 26
V = 100000
H = 128
L = 64

NW = 32
ROWS_PER_W = B // NW
RSUB = 4
GSZ = RSUB * F
NSUB = ROWS_PER_W // RSUB
NLANE = 16


def _gather_sum_body(nsub, rows_per_w, idx_hbm, tables_hbm, out_hbm, idx_v,
                     buf0, buf1, out_v, sem0, sem1, osem):
    NSUB = nsub
    ROWS_PER_W = rows_per_w
    c = lax.axis_index("c")
    s = lax.axis_index("s")
    wid = s * 2 + c

    pltpu.sync_copy(idx_hbm.at[pl.ds(wid * NSUB, NSUB)], idx_v)

    bufs = (buf0, buf1)
    sems = (sem0, sem1)

    pltpu.async_copy(tables_hbm.at[idx_v.at[0]], buf0, sem0)

    def accum(g, buf):
        orow0 = g * RSUB
        for r in range(RSUB):
            for j in range(H // NLANE):
                sl = pl.ds(j * NLANE, NLANE)
                vals = [buf[r * F + f, sl] for f in range(F)]
                while len(vals) > 1:
                    nxt = [vals[i] + vals[i + 1]
                           for i in range(0, len(vals) - 1, 2)]
                    if len(vals) % 2:
                        nxt.append(vals[-1])
                    vals = nxt
                out_v[orow0 + r, sl] = vals[0]

    FLUSH_SUB = 16
    FLUSH_ROWS = FLUSH_SUB * RSUB

    def outer(i, carry):
        for b in range(2):
            g = i * 2 + b
            buf, sem = bufs[b], sems[b]
            nbuf, nsem = bufs[1 - b], sems[1 - b]

            @pl.when(g + 1 < NSUB)
            def _issue():
                pltpu.async_copy(tables_hbm.at[idx_v.at[g + 1]], nbuf, nsem)

            pltpu.make_async_copy(tables_hbm.at[idx_v.at[g]], buf, sem).wait()
            accum(g, buf)

            @pl.when(g % FLUSH_SUB == FLUSH_SUB - 1)
            def _flush():
                row0 = pl.multiple_of((g - (FLUSH_SUB - 1)) * RSUB, FLUSH_ROWS)
                pltpu.async_copy(
                    out_v.at[pl.ds(row0, FLUSH_ROWS)],
                    out_hbm.at[pl.ds(wid * ROWS_PER_W + row0, FLUSH_ROWS)],
                    osem,
                )
        return carry

    lax.fori_loop(0, NSUB // 2, outer, 0)

    def drain(k, carry):
        pltpu.make_async_copy(
            out_v.at[pl.ds(0, FLUSH_ROWS)],
            out_hbm.at[pl.ds(wid * ROWS_PER_W, FLUSH_ROWS)],
            osem,
        ).wait()
        return carry

    lax.fori_loop(0, NSUB // FLUSH_SUB, drain, 0)


@functools.lru_cache(maxsize=None)
def _make_gather_sum(nb):
    rows_per_w = nb // NW
    nsub = rows_per_w // RSUB
    mesh = plsc.VectorSubcoreMesh(core_axis_name="c", subcore_axis_name="s")
    return pl.kernel(
        functools.partial(_gather_sum_body, nsub, rows_per_w),
        out_type=jax.ShapeDtypeStruct((nb, H), jnp.float32),
        mesh=mesh,
        scratch_types=[
            pltpu.VMEM((nsub, GSZ), jnp.int32),
            pltpu.VMEM((GSZ, H), jnp.float32),
            pltpu.VMEM((GSZ, H), jnp.float32),
            pltpu.VMEM((rows_per_w, H), jnp.float32),
            pltpu.SemaphoreType.DMA,
            pltpu.SemaphoreType.DMA,
            pltpu.SemaphoreType.DMA,
        ],
    )


def _gather_sum(idx2, tables2d):
    nb = idx2.shape[0] * RSUB
    return _make_gather_sum(nb)(idx2, tables2d)


def _mlp_body(h_ref, w1_ref, b1_ref, wo_ref, bo_ref, mu_ref, lv_ref):
    h = h_ref[...]
    z = jnp.dot(h, w1_ref[...], preferred_element_type=jnp.float32)
    z = jnp.maximum(z + b1_ref[...], 0.0)
    z2 = (
        jnp.dot(z, wo_ref[...], preferred_element_type=jnp.float32)
        + bo_ref[...]
    )
    mu_ref[...] = z2[:, :L]
    lv_ref[...] = z2[:, L:]


@jax.jit
def _mlp(h, W1, b1, Wo, bo):
    TB = 1024
    nb = h.shape[0]
    grid = (nb // TB,)
    return pl.pallas_call(
        _mlp_body,
        grid=grid,
        in_specs=[
            pl.BlockSpec((TB, H), lambda i: (i, 0)),
            pl.BlockSpec((H, H), lambda i: (0, 0)),
            pl.BlockSpec((1, H), lambda i: (0, 0)),
            pl.BlockSpec((H, 2 * L), lambda i: (0, 0)),
            pl.BlockSpec((1, 2 * L), lambda i: (0, 0)),
        ],
        out_specs=[
            pl.BlockSpec((TB, L), lambda i: (i, 0)),
            pl.BlockSpec((TB, L), lambda i: (i, 0)),
        ],
        out_shape=[
            jax.ShapeDtypeStruct((nb, L), jnp.float32),
            jax.ShapeDtypeStruct((nb, L), jnp.float32),
        ],
    )(h, W1, b1, Wo, bo)


def kernel(x, tables, W1, b1, Wmu, bmu, Wvar, bvar):
    tables2d = tables.reshape(F * V, H)
    offs = jnp.arange(F, dtype=jnp.int32) * V
    idx2 = (x.astype(jnp.int32) + offs[None, :]).reshape(B // RSUB, GSZ)
    Wo = jnp.concatenate([Wmu, Wvar], axis=1)
    bo = jnp.concatenate([bmu, bvar]).reshape(1, 2 * L)
    h = _gather_sum(idx2, tables2d)
    mu, lv = _mlp(h, W1, b1.reshape(1, H), Wo, bo)
    return mu, lv


# speedup vs baseline: 1.5354x; 1.5354x over previous
import functools

import jax
import jax.numpy as jnp
from jax import lax
from jax.experimental import pallas as pl
from jax.experimental.pallas import tpu as pltpu
from jax.experimental.pallas import tpu_sc as plsc

B = 16384
F = 26
V = 100000
H = 128
L = 64

NW = 32
ROWS_PER_W = B // NW
RSUB = 4
GSZ = RSUB * F
NSUB = ROWS_PER_W // RSUB
NLANE = 16


def _gather_sum_body(nsub, rows_per_w, idx_hbm, tables_hbm, out_hbm, idx_v,
                     buf0, buf1, out_v, sem0, sem1, osem):
    NSUB = nsub
    ROWS_PER_W = rows_per_w
    c = lax.axis_index("c")
    s = lax.axis_index("s")
    wid = s * 2 + c

    pltpu.sync_copy(idx_hbm.at[pl.ds(wid * NSUB, NSUB)], idx_v)

    bufs = (buf0, buf1)
    sems = (sem0, sem1)

    pltpu.async_copy(tables_hbm.at[idx_v.at[0]], buf0, sem0)

    def accum(g, buf):
        def row_body(r, carry):
            orow = g * RSUB + r
            for j in range(H // NLANE):
                sl = pl.ds(j * NLANE, NLANE)
                vals = [buf[r * F + f, sl] for f in range(F)]
                while len(vals) > 1:
                    nxt = [vals[i] + vals[i + 1]
                           for i in range(0, len(vals) - 1, 2)]
                    if len(vals) % 2:
                        nxt.append(vals[-1])
                    vals = nxt
                out_v[orow, sl] = vals[0]
            return carry
        lax.fori_loop(0, RSUB, row_body, 0)

    FLUSH_SUB = 16
    FLUSH_ROWS = FLUSH_SUB * RSUB

    def outer(i, carry):
        for b in range(2):
            g = i * 2 + b
            buf, sem = bufs[b], sems[b]
            nbuf, nsem = bufs[1 - b], sems[1 - b]

            @pl.when(g + 1 < NSUB)
            def _issue():
                pltpu.async_copy(tables_hbm.at[idx_v.at[g + 1]], nbuf, nsem)

            pltpu.make_async_copy(tables_hbm.at[idx_v.at[g]], buf, sem).wait()
            accum(g, buf)

            @pl.when(g % FLUSH_SUB == FLUSH_SUB - 1)
            def _flush():
                row0 = pl.multiple_of((g - (FLUSH_SUB - 1)) * RSUB, FLUSH_ROWS)
                pltpu.async_copy(
                    out_v.at[pl.ds(row0, FLUSH_ROWS)],
                    out_hbm.at[pl.ds(wid * ROWS_PER_W + row0, FLUSH_ROWS)],
                    osem,
                )
        return carry

    lax.fori_loop(0, NSUB // 2, outer, 0)

    def drain(k, carry):
        pltpu.make_async_copy(
            out_v.at[pl.ds(0, FLUSH_ROWS)],
            out_hbm.at[pl.ds(wid * ROWS_PER_W, FLUSH_ROWS)],
            osem,
        ).wait()
        return carry

    lax.fori_loop(0, NSUB // FLUSH_SUB, drain, 0)


@functools.lru_cache(maxsize=None)
def _make_gather_sum(nb):
    rows_per_w = nb // NW
    nsub = rows_per_w // RSUB
    mesh = plsc.VectorSubcoreMesh(core_axis_name="c", subcore_axis_name="s")
    return pl.kernel(
        functools.partial(_gather_sum_body, nsub, rows_per_w),
        out_type=jax.ShapeDtypeStruct((nb, H), jnp.float32),
        mesh=mesh,
        scratch_types=[
            pltpu.VMEM((nsub, GSZ), jnp.int32),
            pltpu.VMEM((GSZ, H), jnp.float32),
            pltpu.VMEM((GSZ, H), jnp.float32),
            pltpu.VMEM((rows_per_w, H), jnp.float32),
            pltpu.SemaphoreType.DMA,
            pltpu.SemaphoreType.DMA,
            pltpu.SemaphoreType.DMA,
        ],
    )


def _gather_sum(idx2, tables2d):
    nb = idx2.shape[0] * RSUB
    return _make_gather_sum(nb)(idx2, tables2d)


def _mlp_body(h_ref, w1_ref, b1_ref, wo_ref, bo_ref, mu_ref, lv_ref):
    h = h_ref[...]
    z = jnp.dot(h, w1_ref[...], preferred_element_type=jnp.float32)
    z = jnp.maximum(z + b1_ref[...], 0.0)
    z2 = (
        jnp.dot(z, wo_ref[...], preferred_element_type=jnp.float32)
        + bo_ref[...]
    )
    mu_ref[...] = z2[:, :L]
    lv_ref[...] = z2[:, L:]


@jax.jit
def _mlp(h, W1, b1, Wo, bo):
    TB = 1024
    nb = h.shape[0]
    grid = (nb // TB,)
    return pl.pallas_call(
        _mlp_body,
        grid=grid,
        in_specs=[
            pl.BlockSpec((TB, H), lambda i: (i, 0)),
            pl.BlockSpec((H, H), lambda i: (0, 0)),
            pl.BlockSpec((1, H), lambda i: (0, 0)),
            pl.BlockSpec((H, 2 * L), lambda i: (0, 0)),
            pl.BlockSpec((1, 2 * L), lambda i: (0, 0)),
        ],
        out_specs=[
            pl.BlockSpec((TB, L), lambda i: (i, 0)),
            pl.BlockSpec((TB, L), lambda i: (i, 0)),
        ],
        out_shape=[
            jax.ShapeDtypeStruct((nb, L), jnp.float32),
            jax.ShapeDtypeStruct((nb, L), jnp.float32),
        ],
    )(h, W1, b1, Wo, bo)


def kernel(x, tables, W1, b1, Wmu, bmu, Wvar, bvar):
    tables2d = tables.reshape(F * V, H)
    offs = jnp.arange(F, dtype=jnp.int32) * V
    idx2 = (x.astype(jnp.int32) + offs[None, :]).reshape(B // RSUB, GSZ)
    Wo = jnp.concatenate([Wmu, Wvar], axis=1)
    bo = jnp.concatenate([bmu, bvar]).reshape(1, 2 * L)
    h = _gather_sum(idx2, tables2d)
    mu, lv = _mlp(h, W1, b1.reshape(1, H), Wo, bo)
    return mu, lv


# speedup vs baseline: 1.8478x; 1.2035x over previous
import functools

import jax
import jax.numpy as jnp
from jax import lax
from jax.experimental import pallas as pl
from jax.experimental.pallas import tpu as pltpu
from jax.experimental.pallas import tpu_sc as plsc

B = 16384
F = 26
V = 100000
H = 128
L = 64

NW = 32
RSUB = 4
GSZ = RSUB * F
NLANE = 16
NBUF = 4
FLUSH_SUB = 16
FLUSH_ROWS = FLUSH_SUB * RSUB
ORING_GROUPS = 4
ORING_ROWS = ORING_GROUPS * FLUSH_ROWS


def _gather_sum_body(nsub, rows_per_w, idx_hbm, tables_hbm, out_hbm, idx_v,
                     buf0, buf1, buf2, buf3, out_v, sem0, sem1, sem2, sem3,
                     osem):
    NSUB = nsub
    c = lax.axis_index("c")
    s = lax.axis_index("s")
    wid = s * 2 + c

    pltpu.sync_copy(idx_hbm.at[pl.ds(wid * NSUB, NSUB)], idx_v)

    bufs = (buf0, buf1, buf2, buf3)
    sems = (sem0, sem1, sem2, sem3)

    for p in range(NBUF - 1):
        pltpu.async_copy(tables_hbm.at[idx_v.at[p]], bufs[p], sems[p])

    def accum(g, buf):
        def row_body(r, carry):
            orow = (g % (ORING_GROUPS * FLUSH_SUB)) * RSUB + r
            for j in range(H // NLANE):
                sl = pl.ds(j * NLANE, NLANE)
                vals = [buf[r * F + f, sl] for f in range(F)]
                while len(vals) > 1:
                    nxt = [vals[i] + vals[i + 1]
                           for i in range(0, len(vals) - 1, 2)]
                    if len(vals) % 2:
                        nxt.append(vals[-1])
                    vals = nxt
                out_v[orow, sl] = vals[0]
            return carry
        lax.fori_loop(0, RSUB, row_body, 0)

    def flush_wait():
        pltpu.make_async_copy(
            out_v.at[pl.ds(0, FLUSH_ROWS)],
            out_hbm.at[pl.ds(wid * rows_per_w, FLUSH_ROWS)],
            osem,
        ).wait()

    def outer(i, carry):
        for b in range(NBUF):
            g = i * NBUF + b
            buf, sem = bufs[b], sems[b]
            nb_i = (b + NBUF - 1) % NBUF
            nbuf, nsem = bufs[nb_i], sems[nb_i]

            @pl.when(g + NBUF - 1 < NSUB)
            def _issue():
                pltpu.async_copy(
                    tables_hbm.at[idx_v.at[g + NBUF - 1]], nbuf, nsem)

            pltpu.make_async_copy(tables_hbm.at[idx_v.at[g]], buf, sem).wait()
            accum(g, buf)

            @pl.when(g % FLUSH_SUB == FLUSH_SUB - 1)
            def _flush():
                grp = g // FLUSH_SUB
                ring0 = pl.multiple_of(
                    (grp % ORING_GROUPS) * FLUSH_ROWS, FLUSH_ROWS)
                hbm0 = pl.multiple_of(
                    wid * rows_per_w + (g - (FLUSH_SUB - 1)) * RSUB,
                    FLUSH_ROWS)
                pltpu.async_copy(
                    out_v.at[pl.ds(ring0, FLUSH_ROWS)],
                    out_hbm.at[pl.ds(hbm0, FLUSH_ROWS)],
                    osem,
                )
                @pl.when(grp >= ORING_GROUPS - 1)
                def _retire():
                    flush_wait()
        return carry

    lax.fori_loop(0, NSUB // NBUF, outer, 0)

    def drain(k, carry):
        flush_wait()
        return carry

    lax.fori_loop(0, ORING_GROUPS - 1, drain, 0)


@functools.lru_cache(maxsize=None)
def _make_gather_sum(nb):
    rows_per_w = nb // NW
    nsub = rows_per_w // RSUB
    mesh = plsc.VectorSubcoreMesh(core_axis_name="c", subcore_axis_name="s")
    return pl.kernel(
        functools.partial(_gather_sum_body, nsub, rows_per_w),
        out_type=jax.ShapeDtypeStruct((nb, H), jnp.float32),
        mesh=mesh,
        scratch_types=[
            pltpu.VMEM((nsub, GSZ), jnp.int32),
            pltpu.VMEM((GSZ, H), jnp.float32),
            pltpu.VMEM((GSZ, H), jnp.float32),
            pltpu.VMEM((GSZ, H), jnp.float32),
            pltpu.VMEM((GSZ, H), jnp.float32),
            pltpu.VMEM((ORING_ROWS, H), jnp.float32),
            pltpu.SemaphoreType.DMA,
            pltpu.SemaphoreType.DMA,
            pltpu.SemaphoreType.DMA,
            pltpu.SemaphoreType.DMA,
            pltpu.SemaphoreType.DMA,
        ],
    )


def _gather_sum(idx2, tables2d):
    nb = idx2.shape[0] * RSUB
    return _make_gather_sum(nb)(idx2, tables2d)


def _mlp_body(h_ref, w1_ref, b1_ref, wo_ref, bo_ref, mu_ref, lv_ref):
    h = h_ref[...]
    z = jnp.dot(h, w1_ref[...], preferred_element_type=jnp.float32)
    z = jnp.maximum(z + b1_ref[...], 0.0)
    z2 = (
        jnp.dot(z, wo_ref[...], preferred_element_type=jnp.float32)
        + bo_ref[...]
    )
    mu_ref[...] = z2[:, :L]
    lv_ref[...] = z2[:, L:]


@jax.jit
def _mlp(h, W1, b1, Wo, bo):
    TB = 1024
    nb = h.shape[0]
    grid = (nb // TB,)
    return pl.pallas_call(
        _mlp_body,
        grid=grid,
        in_specs=[
            pl.BlockSpec((TB, H), lambda i: (i, 0)),
            pl.BlockSpec((H, H), lambda i: (0, 0)),
            pl.BlockSpec((1, H), lambda i: (0, 0)),
            pl.BlockSpec((H, 2 * L), lambda i: (0, 0)),
            pl.BlockSpec((1, 2 * L), lambda i: (0, 0)),
        ],
        out_specs=[
            pl.BlockSpec((TB, L), lambda i: (i, 0)),
            pl.BlockSpec((TB, L), lambda i: (i, 0)),
        ],
        out_shape=[
            jax.ShapeDtypeStruct((nb, L), jnp.float32),
            jax.ShapeDtypeStruct((nb, L), jnp.float32),
        ],
    )(h, W1, b1, Wo, bo)


def kernel(x, tables, W1, b1, Wmu, bmu, Wvar, bvar):
    tables2d = tables.reshape(F * V, H)
    offs = jnp.arange(F, dtype=jnp.int32) * V
    idx2 = (x.astype(jnp.int32) + offs[None, :]).reshape(B // RSUB, GSZ)
    Wo = jnp.concatenate([Wmu, Wvar], axis=1)
    bo = jnp.concatenate([bmu, bvar]).reshape(1, 2 * L)
    h = _gather_sum(idx2, tables2d)
    mu, lv = _mlp(h, W1, b1.reshape(1, H), Wo, bo)
    return mu, lv


# speedup vs baseline: 1.9082x; 1.0327x over previous
import functools

import jax
import jax.numpy as jnp
from jax import lax
from jax.experimental import pallas as pl
from jax.experimental.pallas import tpu as pltpu
from jax.experimental.pallas import tpu_sc as plsc

B = 16384
F = 26
V = 100000
H = 128
L = 64

NW = 32
RSUB = 4
GSZ = RSUB * F
NLANE = 16
NBUF = 4
FLUSH_SUB = 16
FLUSH_ROWS = FLUSH_SUB * RSUB
ORING_GROUPS = 4
ORING_ROWS = ORING_GROUPS * FLUSH_ROWS


def _gather_sum_body(nsub, rows_per_w, idx_hbm, tables_hbm, out_hbm, idx_v,
                     buf0, buf1, buf2, buf3, out_v, sem0, sem1, sem2, sem3,
                     osem):
    NSUB = nsub
    c = lax.axis_index("c")
    s = lax.axis_index("s")
    wid = s * 2 + c

    bufs = (buf0, buf1, buf2, buf3)
    sems = (sem0, sem1, sem2, sem3)

    PRE = 8
    pltpu.sync_copy(idx_hbm.at[pl.ds(wid * NSUB, PRE)], idx_v.at[pl.ds(0, PRE)])
    for p in range(NBUF - 1):
        pltpu.async_copy(tables_hbm.at[idx_v.at[p]], bufs[p], sems[p])
    pltpu.sync_copy(idx_hbm.at[pl.ds(wid * NSUB + PRE, NSUB - PRE)],
                    idx_v.at[pl.ds(PRE, NSUB - PRE)])

    def accum(g, buf):
        def row_body(r, carry):
            orow = (g % (ORING_GROUPS * FLUSH_SUB)) * RSUB + r
            for j in range(H // NLANE):
                sl = pl.ds(j * NLANE, NLANE)
                vals = [buf[r * F + f, sl] for f in range(F)]
                while len(vals) > 1:
                    nxt = [vals[i] + vals[i + 1]
                           for i in range(0, len(vals) - 1, 2)]
                    if len(vals) % 2:
                        nxt.append(vals[-1])
                    vals = nxt
                out_v[orow, sl] = vals[0]
            return carry
        lax.fori_loop(0, RSUB, row_body, 0)

    def flush_wait():
        pltpu.make_async_copy(
            out_v.at[pl.ds(0, FLUSH_ROWS)],
            out_hbm.at[pl.ds(wid * rows_per_w, FLUSH_ROWS)],
            osem,
        ).wait()

    def outer(i, carry):
        for b in range(NBUF):
            g = i * NBUF + b
            buf, sem = bufs[b], sems[b]
            nb_i = (b + NBUF - 1) % NBUF
            nbuf, nsem = bufs[nb_i], sems[nb_i]

            @pl.when(g + NBUF - 1 < NSUB)
            def _issue():
                pltpu.async_copy(
                    tables_hbm.at[idx_v.at[g + NBUF - 1]], nbuf, nsem)

            pltpu.make_async_copy(tables_hbm.at[idx_v.at[g]], buf, sem).wait()
            accum(g, buf)

            @pl.when(g % FLUSH_SUB == FLUSH_SUB - 1)
            def _flush():
                grp = g // FLUSH_SUB
                ring0 = pl.multiple_of(
                    (grp % ORING_GROUPS) * FLUSH_ROWS, FLUSH_ROWS)
                hbm0 = pl.multiple_of(
                    wid * rows_per_w + (g - (FLUSH_SUB - 1)) * RSUB,
                    FLUSH_ROWS)
                pltpu.async_copy(
                    out_v.at[pl.ds(ring0, FLUSH_ROWS)],
                    out_hbm.at[pl.ds(hbm0, FLUSH_ROWS)],
                    osem,
                )
                @pl.when(grp >= ORING_GROUPS - 1)
                def _retire():
                    flush_wait()
        return carry

    lax.fori_loop(0, NSUB // NBUF, outer, 0)

    def drain(k, carry):
        flush_wait()
        return carry

    lax.fori_loop(0, ORING_GROUPS - 1, drain, 0)


@functools.lru_cache(maxsize=None)
def _make_gather_sum(nb):
    rows_per_w = nb // NW
    nsub = rows_per_w // RSUB
    mesh = plsc.VectorSubcoreMesh(core_axis_name="c", subcore_axis_name="s")
    return pl.kernel(
        functools.partial(_gather_sum_body, nsub, rows_per_w),
        out_type=jax.ShapeDtypeStruct((nb, H), jnp.float32),
        mesh=mesh,
        scratch_types=[
            pltpu.VMEM((nsub, GSZ), jnp.int32),
            pltpu.VMEM((GSZ, H), jnp.float32),
            pltpu.VMEM((GSZ, H), jnp.float32),
            pltpu.VMEM((GSZ, H), jnp.float32),
            pltpu.VMEM((GSZ, H), jnp.float32),
            pltpu.VMEM((ORING_ROWS, H), jnp.float32),
            pltpu.SemaphoreType.DMA,
            pltpu.SemaphoreType.DMA,
            pltpu.SemaphoreType.DMA,
            pltpu.SemaphoreType.DMA,
            pltpu.SemaphoreType.DMA,
        ],
    )


def _gather_sum(idx2, tables2d):
    nb = idx2.shape[0] * RSUB
    return _make_gather_sum(nb)(idx2, tables2d)


def _mlp_body(h_ref, w1_ref, b1_ref, wo_ref, bo_ref, mu_ref, lv_ref):
    h = h_ref[...]
    z = jnp.dot(h, w1_ref[...], preferred_element_type=jnp.float32)
    z = jnp.maximum(z + b1_ref[...], 0.0)
    z2 = (
        jnp.dot(z, wo_ref[...], preferred_element_type=jnp.float32)
        + bo_ref[...]
    )
    mu_ref[...] = z2[:, :L]
    lv_ref[...] = z2[:, L:]


@jax.jit
def _mlp(h, W1, b1, Wo, bo):
    TB = 2048
    nb = h.shape[0]
    grid = (nb // TB,)
    return pl.pallas_call(
        _mlp_body,
        grid=grid,
        in_specs=[
            pl.BlockSpec((TB, H), lambda i: (i, 0)),
            pl.BlockSpec((H, H), lambda i: (0, 0)),
            pl.BlockSpec((1, H), lambda i: (0, 0)),
            pl.BlockSpec((H, 2 * L), lambda i: (0, 0)),
            pl.BlockSpec((1, 2 * L), lambda i: (0, 0)),
        ],
        out_specs=[
            pl.BlockSpec((TB, L), lambda i: (i, 0)),
            pl.BlockSpec((TB, L), lambda i: (i, 0)),
        ],
        out_shape=[
            jax.ShapeDtypeStruct((nb, L), jnp.float32),
            jax.ShapeDtypeStruct((nb, L), jnp.float32),
        ],
    )(h, W1, b1, Wo, bo)


def kernel(x, tables, W1, b1, Wmu, bmu, Wvar, bvar):
    tables2d = tables.reshape(F * V, H)
    offs = jnp.arange(F, dtype=jnp.int32) * V
    idx2 = (x.astype(jnp.int32) + offs[None, :]).reshape(B // RSUB, GSZ)
    Wo = jnp.concatenate([Wmu, Wvar], axis=1)
    bo = jnp.concatenate([bmu, bvar]).reshape(1, 2 * L)
    h = _gather_sum(idx2, tables2d)
    mu, lv = _mlp(h, W1, b1.reshape(1, H), Wo, bo)
    return mu, lv
